# baseline (speedup 1.0000x reference)
import jax
import jax.numpy as jnp
from jax.experimental import pallas as pl
from jax.experimental.pallas import tpu as pltpu

N = 10000
D = 128
BM_AUG = 200
BM_ADJ = 200


def _aug_kernel(seq1_ref, seq2_ref, w_ref, aug1_ref, aug2_ref, bias_ref,
                a_ref, f1_ref, f2_ref, sums_ref):
    i = pl.program_id(0)

    @pl.when(i == 0)
    def _():
        w = w_ref[...]
        f1_ref[...] = jnp.dot(seq1_ref[...], w,
                              preferred_element_type=jnp.float32).astype(jnp.bfloat16)
        f2_ref[...] = jnp.dot(seq2_ref[...], w,
                              preferred_element_type=jnp.float32).astype(jnp.bfloat16)
        sums_ref[...] = jnp.zeros_like(sums_ref)

    @pl.when(i > 0)
    def _():
        f1 = f1_ref[...]
        a = a_ref[0, 0]
        b = bias_ref[...]
        g1 = jnp.dot(aug1_ref[...].astype(jnp.bfloat16), f1,
                     preferred_element_type=jnp.float32) + b
        g3 = jnp.dot(aug2_ref[...].astype(jnp.bfloat16), f1,
                     preferred_element_type=jnp.float32) + b
        h1 = jnp.where(g1 >= 0, g1, a * g1)
        h3 = jnp.where(g3 >= 0, g3, a * g3)
        sums_ref[0:1, :] += jnp.sum(h1, axis=0, keepdims=True)
        sums_ref[1:2, :] += jnp.sum(h3, axis=0, keepdims=True)


def _adj_kernel(sums_ref, adj_ref, f1_ref, f2_ref, bias_ref, a_ref, wbt_ref,
                bb_ref, out_ref):
    adj_blk = adj_ref[...].astype(jnp.bfloat16)
    a = a_ref[0, 0]
    b = bias_ref[...]
    g0 = jnp.dot(adj_blk, f1_ref[...], preferred_element_type=jnp.float32) + b
    g2 = jnp.dot(adj_blk, f2_ref[...], preferred_element_type=jnp.float32) + b
    h0 = jnp.where(g0 >= 0, g0, a * g0)
    h2 = jnp.where(g2 >= 0, g2, a * g2)
    c1 = jax.nn.sigmoid(sums_ref[0:1, :] / N)
    c3 = jax.nn.sigmoid(sums_ref[1:2, :] / N)
    v = jnp.dot(c1 + c3, wbt_ref[...], preferred_element_type=jnp.float32)
    two_bb = 2.0 * bb_ref[0, 0]
    out_ref[:, 0:1] = jnp.sum(h0 * v, axis=1, keepdims=True) + two_bb
    out_ref[:, 1:2] = jnp.sum(h2 * v, axis=1, keepdims=True) + two_bb


@jax.jit
def kernel(seq1, seq2, adj, aug_adj1, aug_adj2, W, bias, prelu_a, Wb, bb):
    bias2 = bias.reshape(1, D)
    a2 = jnp.reshape(prelu_a, (1, 1))
    bb2 = jnp.reshape(bb, (1, 1))

    nb = N // BM_AUG
    f1, f2, sums = pl.pallas_call(
        _aug_kernel,
        grid=(nb + 1,),
        in_specs=[
            pl.BlockSpec((N, D), lambda i: (0, 0)),
            pl.BlockSpec((N, D), lambda i: (0, 0)),
            pl.BlockSpec((D, D), lambda i: (0, 0)),
            pl.BlockSpec((BM_AUG, N), lambda i: (jnp.maximum(i - 1, 0), 0)),
            pl.BlockSpec((BM_AUG, N), lambda i: (jnp.maximum(i - 1, 0), 0)),
            pl.BlockSpec((1, D), lambda i: (0, 0)),
            pl.BlockSpec((1, 1), lambda i: (0, 0)),
        ],
        out_specs=[pl.BlockSpec((N, D), lambda i: (0, 0)),
                   pl.BlockSpec((N, D), lambda i: (0, 0)),
                   pl.BlockSpec((2, D), lambda i: (0, 0))],
        out_shape=[jax.ShapeDtypeStruct((N, D), jnp.bfloat16),
                   jax.ShapeDtypeStruct((N, D), jnp.bfloat16),
                   jax.ShapeDtypeStruct((2, D), jnp.float32)],
        compiler_params=pltpu.CompilerParams(vmem_limit_bytes=65_000_000),
    )(seq1[0], seq2[0], W, aug_adj1, aug_adj2, bias2, a2)

    out2 = pl.pallas_call(
        _adj_kernel,
        grid=(N // BM_ADJ,),
        in_specs=[
            pl.BlockSpec((2, D), lambda i: (0, 0)),
            pl.BlockSpec((BM_ADJ, N), lambda i: (i, 0)),
            pl.BlockSpec((N, D), lambda i: (0, 0)),
            pl.BlockSpec((N, D), lambda i: (0, 0)),
            pl.BlockSpec((1, D), lambda i: (0, 0)),
            pl.BlockSpec((1, 1), lambda i: (0, 0)),
            pl.BlockSpec((D, D), lambda i: (0, 0)),
            pl.BlockSpec((1, 1), lambda i: (0, 0)),
        ],
        out_specs=pl.BlockSpec((BM_ADJ, 2), lambda i: (i, 0)),
        out_shape=jax.ShapeDtypeStruct((N, 2), jnp.float32),
        compiler_params=pltpu.CompilerParams(vmem_limit_bytes=65_000_000),
    )(sums, adj, f1, f2, bias2, a2, Wb[0].T, bb2)

    ret = jnp.concatenate([out2[:, 0], out2[:, 1]])[None, :]
    return ret


# speedup vs baseline: 1.0420x; 1.0420x over previous
import jax
import jax.numpy as jnp
from jax.experimental import pallas as pl
from jax.experimental.pallas import tpu as pltpu

N = 10000
D = 128
BM_AUG = 200
BM_ADJ = 400


def _aug_kernel(seq1_ref, seq2_ref, w_ref, aug1_ref, aug2_ref, bias_ref,
                a_ref, f1_ref, f2_ref, sums_ref):
    i = pl.program_id(0)

    @pl.when(i == 0)
    def _():
        w = w_ref[...]
        f1_ref[...] = jnp.dot(seq1_ref[...], w,
                              preferred_element_type=jnp.float32).astype(jnp.bfloat16)
        f2_ref[...] = jnp.dot(seq2_ref[...], w,
                              preferred_element_type=jnp.float32).astype(jnp.bfloat16)
        sums_ref[...] = jnp.zeros_like(sums_ref)

    @pl.when(i > 0)
    def _():
        f1 = f1_ref[...]
        a = a_ref[0, 0]
        b = bias_ref[...]
        g1 = jnp.dot(aug1_ref[...].astype(jnp.bfloat16), f1,
                     preferred_element_type=jnp.float32) + b
        g3 = jnp.dot(aug2_ref[...].astype(jnp.bfloat16), f1,
                     preferred_element_type=jnp.float32) + b
        h1 = jnp.where(g1 >= 0, g1, a * g1)
        h3 = jnp.where(g3 >= 0, g3, a * g3)
        sums_ref[0:1, :] += jnp.sum(h1, axis=0, keepdims=True)
        sums_ref[1:2, :] += jnp.sum(h3, axis=0, keepdims=True)


def _adj_kernel(sums_ref, adj_ref, f1_ref, f2_ref, bias_ref, a_ref, wbt_ref,
                bb_ref, out_ref):
    adj_blk = adj_ref[...].astype(jnp.bfloat16)
    a = a_ref[0, 0]
    b = bias_ref[...]
    g0 = jnp.dot(adj_blk, f1_ref[...], preferred_element_type=jnp.float32) + b
    g2 = jnp.dot(adj_blk, f2_ref[...], preferred_element_type=jnp.float32) + b
    h0 = jnp.where(g0 >= 0, g0, a * g0)
    h2 = jnp.where(g2 >= 0, g2, a * g2)
    c1 = jax.nn.sigmoid(sums_ref[0:1, :] / N)
    c3 = jax.nn.sigmoid(sums_ref[1:2, :] / N)
    v = jnp.dot(c1 + c3, wbt_ref[...], preferred_element_type=jnp.float32)
    two_bb = 2.0 * bb_ref[0, 0]
    out_ref[:, 0:1] = jnp.sum(h0 * v, axis=1, keepdims=True) + two_bb
    out_ref[:, 1:2] = jnp.sum(h2 * v, axis=1, keepdims=True) + two_bb


@jax.jit
def kernel(seq1, seq2, adj, aug_adj1, aug_adj2, W, bias, prelu_a, Wb, bb):
    bias2 = bias.reshape(1, D)
    a2 = jnp.reshape(prelu_a, (1, 1))
    bb2 = jnp.reshape(bb, (1, 1))

    nb = N // BM_AUG
    f1, f2, sums = pl.pallas_call(
        _aug_kernel,
        grid=(nb + 1,),
        in_specs=[
            pl.BlockSpec((N, D), lambda i: (0, 0)),
            pl.BlockSpec((N, D), lambda i: (0, 0)),
            pl.BlockSpec((D, D), lambda i: (0, 0)),
            pl.BlockSpec((BM_AUG, N), lambda i: (jnp.maximum(i - 1, 0), 0)),
            pl.BlockSpec((BM_AUG, N), lambda i: (jnp.maximum(i - 1, 0), 0)),
            pl.BlockSpec((1, D), lambda i: (0, 0)),
            pl.BlockSpec((1, 1), lambda i: (0, 0)),
        ],
        out_specs=[pl.BlockSpec((N, D), lambda i: (0, 0)),
                   pl.BlockSpec((N, D), lambda i: (0, 0)),
                   pl.BlockSpec((2, D), lambda i: (0, 0))],
        out_shape=[jax.ShapeDtypeStruct((N, D), jnp.bfloat16),
                   jax.ShapeDtypeStruct((N, D), jnp.bfloat16),
                   jax.ShapeDtypeStruct((2, D), jnp.float32)],
        compiler_params=pltpu.CompilerParams(vmem_limit_bytes=65_000_000),
    )(seq1[0], seq2[0], W, aug_adj1, aug_adj2, bias2, a2)

    out2 = pl.pallas_call(
        _adj_kernel,
        grid=(N // BM_ADJ,),
        in_specs=[
            pl.BlockSpec((2, D), lambda i: (0, 0)),
            pl.BlockSpec((BM_ADJ, N), lambda i: (i, 0)),
            pl.BlockSpec((N, D), lambda i: (0, 0)),
            pl.BlockSpec((N, D), lambda i: (0, 0)),
            pl.BlockSpec((1, D), lambda i: (0, 0)),
            pl.BlockSpec((1, 1), lambda i: (0, 0)),
            pl.BlockSpec((D, D), lambda i: (0, 0)),
            pl.BlockSpec((1, 1), lambda i: (0, 0)),
        ],
        out_specs=pl.BlockSpec((BM_ADJ, 2), lambda i: (i, 0)),
        out_shape=jax.ShapeDtypeStruct((N, 2), jnp.float32),
        compiler_params=pltpu.CompilerParams(
            vmem_limit_bytes=65_000_000,
            dimension_semantics=("parallel",)),
    )(sums, adj, f1, f2, bias2, a2, Wb[0].T, bb2)

    ret = jnp.concatenate([out2[:, 0], out2[:, 1]])[None, :]
    return ret
